# initial kernel scaffold (unmeasured)
import jax
import jax.numpy as jnp
from jax import lax
from jax.experimental import pallas as pl
from jax.experimental.pallas import tpu as pltpu

N_DEV = 4
SQ = 2048
SKV_LOC = 2048
SKV = 8192
HL = 8
DH = 128
DM = 1024
SCALE = 0.08838834764831843
QT = 128
NQT = SQ // QT
QB = 64


def _attn_body(q_ref, kt_ref, vt_ref, ctx_ref,
               k_all, v_all, send_sems, recv_sems, copy_sems):
    my = lax.axis_index("i")

    bsem = pltpu.get_barrier_semaphore()
    for o in (1, 2, 3):
        pl.semaphore_signal(
            bsem, inc=1,
            device_id=((my + o) % N_DEV,),
            device_id_type=pl.DeviceIdType.MESH,
        )
    pl.semaphore_wait(bsem, 3)

    kc = pltpu.make_async_copy(kt_ref.at[my], k_all.at[my], copy_sems.at[0])
    vc = pltpu.make_async_copy(vt_ref.at[my], v_all.at[my], copy_sems.at[1])
    kc.start()
    vc.start()

    rdmas = []
    for o in (1, 2, 3):
        peer = (my + o) % N_DEV
        for t, (src, dst) in enumerate(((kt_ref, k_all), (vt_ref, v_all))):
            r = pltpu.make_async_remote_copy(
                src_ref=src.at[peer],
                dst_ref=dst.at[my],
                send_sem=send_sems.at[t, o - 1],
                recv_sem=recv_sems.at[t, o - 1],
                device_id=(peer,),
                device_id_type=pl.DeviceIdType.MESH,
            )
            r.start()
            rdmas.append(r)

    kc.wait()
    vc.wait()
    for r in rdmas:
        r.wait()

    qb_ = lax.broadcasted_iota(jnp.int32, (SQ // QB, SKV), 0)
    kb_ = lax.broadcasted_iota(jnp.int32, (SQ // QB, SKV), 1) // QB
    keep = (qb_ == kb_) | (kb_ == 0) | (((qb_ + kb_) % 3) == 0)
    bias_all = jnp.where(keep, 0.0, -1e9).astype(jnp.float32)

    for h in range(HL):
        kh = k_all[:, h].reshape(SKV, DH)
        vh = v_all[:, h].reshape(SKV, DH)

        def qt_step(qt, _, kh=kh, vh=vh, h=h):
            q = q_ref[pl.ds(qt * QT, QT), pl.ds(h * DH, DH)]
            s = lax.dot_general(
                q, kh, (((1,), (1,)), ((), ())),
                preferred_element_type=jnp.float32,
            )
            for i in range(QT // QB):
                bias = lax.dynamic_slice(
                    bias_all, (qt * (QT // QB) + i, 0), (1, SKV))
                sh = s[i * QB:(i + 1) * QB] + bias
                m = jnp.max(sh, axis=1, keepdims=True)
                w = jnp.exp(sh - m)
                l = jnp.sum(w, axis=1, keepdims=True)
                p = (w / l).astype(jnp.bfloat16)
                c = jnp.dot(p, vh, preferred_element_type=jnp.float32)
                ctx_ref[pl.ds(qt * QT + i * QB, QB), pl.ds(h * DH, DH)] = (
                    c.astype(jnp.bfloat16))
            return 0

        lax.fori_loop(0, NQT, qt_step, 0)


def _out_body(ctx_ref, wo_ref, out_ref, comm, send_sems, recv_sems):
    my = lax.axis_index("i")
    left = (my - 1) % N_DEV
    right = (my + 1) % N_DEV

    par = jnp.dot(ctx_ref[...], wo_ref[...],
                  preferred_element_type=jnp.float32)

    bsem = pltpu.get_barrier_semaphore()
    for nbr in (left, right):
        pl.semaphore_signal(
            bsem, inc=1,
            device_id=(nbr,), device_id_type=pl.DeviceIdType.MESH,
        )
    pl.semaphore_wait(bsem, 2)

    out_ref[...] = par
    comm[0] = par.astype(jnp.bfloat16)

    for hop in range(N_DEV - 1):
        s_slot = hop % 2
        r_slot = (hop + 1) % 2
        r = pltpu.make_async_remote_copy(
            src_ref=comm.at[s_slot],
            dst_ref=comm.at[r_slot],
            send_sem=send_sems.at[s_slot],
            recv_sem=recv_sems.at[r_slot],
            device_id=(right,),
            device_id_type=pl.DeviceIdType.MESH,
        )
        r.start()
        r.wait()
        out_ref[...] = out_ref[...] + comm[r_slot].astype(jnp.float32)


def kernel(x, Wq, K_ext, V_ext, Wo):
    xb = x[0].astype(jnp.bfloat16)
    wqb = Wq.astype(jnp.bfloat16)
    q = (jnp.dot(xb, wqb) * SCALE).astype(jnp.bfloat16)

    kt = (K_ext[0].astype(jnp.bfloat16)
          .reshape(SKV_LOC, N_DEV, HL, DH).transpose(1, 2, 0, 3))
    vt = (V_ext[0].astype(jnp.bfloat16)
          .reshape(SKV_LOC, N_DEV, HL, DH).transpose(1, 2, 0, 3))
    wob = Wo.astype(jnp.bfloat16)

    ctx = pl.pallas_call(
        _attn_body,
        out_shape=jax.ShapeDtypeStruct((SQ, HL * DH), jnp.bfloat16),
        in_specs=[
            pl.BlockSpec(memory_space=pltpu.VMEM),
            pl.BlockSpec(memory_space=pltpu.ANY),
            pl.BlockSpec(memory_space=pltpu.ANY),
        ],
        out_specs=pl.BlockSpec(memory_space=pltpu.VMEM),
        scratch_shapes=[
            pltpu.VMEM((N_DEV, HL, SKV_LOC, DH), jnp.bfloat16),
            pltpu.VMEM((N_DEV, HL, SKV_LOC, DH), jnp.bfloat16),
            pltpu.SemaphoreType.DMA((2, 3)),
            pltpu.SemaphoreType.DMA((2, 3)),
            pltpu.SemaphoreType.DMA((2,)),
        ],
        compiler_params=pltpu.CompilerParams(collective_id=0),
    )(q, kt, vt)

    out = pl.pallas_call(
        _out_body,
        out_shape=jax.ShapeDtypeStruct((SQ, DM), jnp.float32),
        in_specs=[
            pl.BlockSpec(memory_space=pltpu.VMEM),
            pl.BlockSpec(memory_space=pltpu.VMEM),
        ],
        out_specs=pl.BlockSpec(memory_space=pltpu.VMEM),
        scratch_shapes=[
            pltpu.VMEM((2, SQ, DM), jnp.bfloat16),
            pltpu.SemaphoreType.DMA((2,)),
            pltpu.SemaphoreType.DMA((2,)),
        ],
        compiler_params=pltpu.CompilerParams(collective_id=1),
    )(ctx, wob)

    return out[None]


# baseline (device time: 736147 ns/iter reference)
import jax
import jax.numpy as jnp
from jax import lax
from jax.experimental import pallas as pl
from jax.experimental.pallas import tpu as pltpu

N_DEV = 4
SQ = 2048
SKV_LOC = 2048
SKV = 8192
HL = 8
DH = 128
DM = 1024
SCALE = 0.08838834764831843
QT = 128
NQT = SQ // QT
QB = 64


def _attn_body(q_ref, kt_ref, vt_ref, ctx_ref,
               k_all, v_all, bias_ref, send_sems, recv_sems, copy_sems):
    my = lax.axis_index("i")

    bsem = pltpu.get_barrier_semaphore()
    for o in (1, 2, 3):
        pl.semaphore_signal(
            bsem, inc=1,
            device_id=((my + o) % N_DEV,),
            device_id_type=pl.DeviceIdType.MESH,
        )
    pl.semaphore_wait(bsem, 3)

    kc = pltpu.make_async_copy(kt_ref.at[my], k_all.at[my], copy_sems.at[0])
    vc = pltpu.make_async_copy(vt_ref.at[my], v_all.at[my], copy_sems.at[1])
    kc.start()
    vc.start()

    rdmas = []
    for o in (1, 2, 3):
        peer = (my + o) % N_DEV
        for t, (src, dst) in enumerate(((kt_ref, k_all), (vt_ref, v_all))):
            r = pltpu.make_async_remote_copy(
                src_ref=src.at[peer],
                dst_ref=dst.at[my],
                send_sem=send_sems.at[t, o - 1],
                recv_sem=recv_sems.at[t, o - 1],
                device_id=(peer,),
                device_id_type=pl.DeviceIdType.MESH,
            )
            r.start()
            rdmas.append(r)

    kc.wait()
    vc.wait()
    for r in rdmas:
        r.wait()

    qb_ = lax.broadcasted_iota(jnp.int32, (SQ // QB, SKV), 0)
    kb_ = lax.broadcasted_iota(jnp.int32, (SQ // QB, SKV), 1) // QB
    keep = (qb_ == kb_) | (kb_ == 0) | (((qb_ + kb_) % 3) == 0)
    bias_ref[...] = jnp.where(keep, 0.0, -1e9).astype(jnp.float32)

    for h in range(HL):
        kh = k_all[:, h].reshape(SKV, DH)
        vh = v_all[:, h].reshape(SKV, DH)

        def qt_step(qt, _, kh=kh, vh=vh, h=h):
            q = q_ref[pl.ds(qt * QT, QT), pl.ds(h * DH, DH)]
            s = lax.dot_general(
                q, kh, (((1,), (1,)), ((), ())),
                preferred_element_type=jnp.float32,
            )
            for i in range(QT // QB):
                bias = bias_ref[pl.ds(qt * (QT // QB) + i, 1), :]
                sh = s[i * QB:(i + 1) * QB] + bias
                m = jnp.max(sh, axis=1, keepdims=True)
                w = jnp.exp(sh - m)
                l = jnp.sum(w, axis=1, keepdims=True)
                p = (w / l).astype(jnp.bfloat16)
                c = jnp.dot(p, vh, preferred_element_type=jnp.float32)
                ctx_ref[pl.ds(qt * QT + i * QB, QB), pl.ds(h * DH, DH)] = (
                    c.astype(jnp.bfloat16))
            return 0

        lax.fori_loop(0, NQT, qt_step, 0)


def _out_body(ctx_ref, wo_ref, out_ref, comm, send_sems, recv_sems):
    my = lax.axis_index("i")
    left = (my - 1) % N_DEV
    right = (my + 1) % N_DEV

    par = jnp.dot(ctx_ref[...], wo_ref[...],
                  preferred_element_type=jnp.float32)

    bsem = pltpu.get_barrier_semaphore()
    for nbr in (left, right):
        pl.semaphore_signal(
            bsem, inc=1,
            device_id=(nbr,), device_id_type=pl.DeviceIdType.MESH,
        )
    pl.semaphore_wait(bsem, 2)

    out_ref[...] = par
    comm[0] = par.astype(jnp.bfloat16)

    for hop in range(N_DEV - 1):
        s_slot = hop % 2
        r_slot = (hop + 1) % 2
        r = pltpu.make_async_remote_copy(
            src_ref=comm.at[s_slot],
            dst_ref=comm.at[r_slot],
            send_sem=send_sems.at[s_slot],
            recv_sem=recv_sems.at[r_slot],
            device_id=(right,),
            device_id_type=pl.DeviceIdType.MESH,
        )
        r.start()
        r.wait()
        out_ref[...] = out_ref[...] + comm[r_slot].astype(jnp.float32)


def kernel(x, Wq, K_ext, V_ext, Wo):
    xb = x[0].astype(jnp.bfloat16)
    wqb = Wq.astype(jnp.bfloat16)
    q = (jnp.dot(xb, wqb) * SCALE).astype(jnp.bfloat16)

    kt = (K_ext[0].astype(jnp.bfloat16)
          .reshape(SKV_LOC, N_DEV, HL, DH).transpose(1, 2, 0, 3))
    vt = (V_ext[0].astype(jnp.bfloat16)
          .reshape(SKV_LOC, N_DEV, HL, DH).transpose(1, 2, 0, 3))
    wob = Wo.astype(jnp.bfloat16)

    ctx = pl.pallas_call(
        _attn_body,
        out_shape=jax.ShapeDtypeStruct((SQ, HL * DH), jnp.bfloat16),
        in_specs=[
            pl.BlockSpec(memory_space=pltpu.VMEM),
            pl.BlockSpec(memory_space=pl.ANY),
            pl.BlockSpec(memory_space=pl.ANY),
        ],
        out_specs=pl.BlockSpec(memory_space=pltpu.VMEM),
        scratch_shapes=[
            pltpu.VMEM((N_DEV, HL, SKV_LOC, DH), jnp.bfloat16),
            pltpu.VMEM((N_DEV, HL, SKV_LOC, DH), jnp.bfloat16),
            pltpu.VMEM((SQ // QB, SKV), jnp.float32),
            pltpu.SemaphoreType.DMA((2, 3)),
            pltpu.SemaphoreType.DMA((2, 3)),
            pltpu.SemaphoreType.DMA((2,)),
        ],
        compiler_params=pltpu.CompilerParams(
            collective_id=0, vmem_limit_bytes=60 * 1024 * 1024),
    )(q, kt, vt)

    out = pl.pallas_call(
        _out_body,
        out_shape=jax.ShapeDtypeStruct((SQ, DM), jnp.float32),
        in_specs=[
            pl.BlockSpec(memory_space=pltpu.VMEM),
            pl.BlockSpec(memory_space=pltpu.VMEM),
        ],
        out_specs=pl.BlockSpec(memory_space=pltpu.VMEM),
        scratch_shapes=[
            pltpu.VMEM((2, SQ, DM), jnp.bfloat16),
            pltpu.SemaphoreType.DMA((2,)),
            pltpu.SemaphoreType.DMA((2,)),
        ],
        compiler_params=pltpu.CompilerParams(
            collective_id=1, vmem_limit_bytes=60 * 1024 * 1024),
    )(ctx, wob)

    return out[None]


# device time: 515804 ns/iter; 1.4272x vs baseline; 1.4272x over previous
import jax
import jax.numpy as jnp
from jax import lax
from jax.experimental import pallas as pl
from jax.experimental.pallas import tpu as pltpu

N_DEV = 4
SQ = 2048
SKV_LOC = 2048
SKV = 8192
HL = 8
DH = 128
DM = 1024
SCALE = 0.08838834764831843
QT = 128
NQT = SQ // QT
QB = 64


def _attn_body(q_ref, kt_ref, vt_ref, ctx_ref,
               k_all, v_all, bias_ref, send_sems, recv_sems, copy_sems):
    my = lax.axis_index("i")

    bsem = pltpu.get_barrier_semaphore()
    for o in (1, 2, 3):
        pl.semaphore_signal(
            bsem, inc=1,
            device_id=((my + o) % N_DEV,),
            device_id_type=pl.DeviceIdType.MESH,
        )
    pl.semaphore_wait(bsem, 3)

    kc = pltpu.make_async_copy(kt_ref.at[my], k_all.at[my], copy_sems.at[0])
    vc = pltpu.make_async_copy(vt_ref.at[my], v_all.at[my], copy_sems.at[1])
    kc.start()
    vc.start()

    per_head = [[] for _ in range(HL)]
    for h in range(HL):
        for o in (1, 2, 3):
            peer = (my + o) % N_DEV
            for t, (src, dst) in enumerate(((kt_ref, k_all), (vt_ref, v_all))):
                r = pltpu.make_async_remote_copy(
                    src_ref=src.at[peer, h],
                    dst_ref=dst.at[my, h],
                    send_sem=send_sems.at[t, o - 1, h],
                    recv_sem=recv_sems.at[t, o - 1, h],
                    device_id=(peer,),
                    device_id_type=pl.DeviceIdType.MESH,
                )
                r.start()
                per_head[h].append(r)

    qb_ = lax.broadcasted_iota(jnp.int32, (SQ // QB, SKV), 0)
    kb_ = lax.broadcasted_iota(jnp.int32, (SQ // QB, SKV), 1) // QB
    keep = (qb_ == kb_) | (kb_ == 0) | (((qb_ + kb_) % 3) == 0)
    bias_ref[...] = jnp.where(keep, 0.0, -1e9).astype(jnp.float32)

    kc.wait()
    vc.wait()
    for h in range(HL):
        for r in per_head[h]:
            r.wait()
        kh = k_all[:, h].reshape(SKV, DH)
        vh = v_all[:, h].reshape(SKV, DH)

        def qt_step(qt, _, kh=kh, vh=vh, h=h):
            q = q_ref[pl.ds(qt * QT, QT), pl.ds(h * DH, DH)]
            s = lax.dot_general(
                q, kh, (((1,), (1,)), ((), ())),
                preferred_element_type=jnp.float32,
            )
            for i in range(QT // QB):
                bias = bias_ref[pl.ds(qt * (QT // QB) + i, 1), :]
                sh = s[i * QB:(i + 1) * QB] + bias
                m = jnp.max(sh, axis=1, keepdims=True)
                w = jnp.exp(sh - m)
                l = jnp.sum(w, axis=1, keepdims=True)
                p = (w / l).astype(jnp.bfloat16)
                c = jnp.dot(p, vh, preferred_element_type=jnp.float32)
                ctx_ref[pl.ds(qt * QT + i * QB, QB), pl.ds(h * DH, DH)] = (
                    c.astype(jnp.bfloat16))
            return 0

        lax.fori_loop(0, NQT, qt_step, 0)


QR = SQ // N_DEV


def _out_body(ctx_ref, wo_ref, out_ref, par_ref, comm, send_sems, recv_sems):
    my = lax.axis_index("i")
    left = (my - 1) % N_DEV
    right = (my + 1) % N_DEV

    par_ref[...] = jnp.dot(ctx_ref[...], wo_ref[...],
                           preferred_element_type=jnp.float32)

    bsem = pltpu.get_barrier_semaphore()
    for nbr in (left, right):
        pl.semaphore_signal(
            bsem, inc=1,
            device_id=(nbr,), device_id_type=pl.DeviceIdType.MESH,
        )
    pl.semaphore_wait(bsem, 2)

    comm[0] = par_ref[pl.ds(my * QR, QR), :].astype(jnp.bfloat16)

    def hop(n):
        s_slot = n % 2
        r_slot = (n + 1) % 2
        r = pltpu.make_async_remote_copy(
            src_ref=comm.at[s_slot],
            dst_ref=comm.at[r_slot],
            send_sem=send_sems.at[s_slot],
            recv_sem=recv_sems.at[r_slot],
            device_id=(right,),
            device_id_type=pl.DeviceIdType.MESH,
        )
        r.start()
        r.wait()
        return r_slot

    for t in range(N_DEV - 1):
        r_slot = hop(t)
        rq = (my - 1 - t) % N_DEV
        val = comm[r_slot].astype(jnp.float32) + par_ref[pl.ds(rq * QR, QR), :]
        comm[r_slot] = val.astype(jnp.bfloat16)
        if t == N_DEV - 2:
            out_ref[pl.ds(((my + 1) % N_DEV) * QR, QR), :] = val

    for u in range(N_DEV - 1):
        r_slot = hop(N_DEV - 1 + u)
        q_idx = (my - u) % N_DEV
        out_ref[pl.ds(q_idx * QR, QR), :] = comm[r_slot].astype(jnp.float32)


def kernel(x, Wq, K_ext, V_ext, Wo):
    xb = x[0].astype(jnp.bfloat16)
    wqb = Wq.astype(jnp.bfloat16)
    q = (jnp.dot(xb, wqb) * SCALE).astype(jnp.bfloat16)

    kt = (K_ext[0].astype(jnp.bfloat16)
          .reshape(SKV_LOC, N_DEV, HL, DH).transpose(1, 2, 0, 3))
    vt = (V_ext[0].astype(jnp.bfloat16)
          .reshape(SKV_LOC, N_DEV, HL, DH).transpose(1, 2, 0, 3))
    wob = Wo.astype(jnp.bfloat16)

    ctx = pl.pallas_call(
        _attn_body,
        out_shape=jax.ShapeDtypeStruct((SQ, HL * DH), jnp.bfloat16),
        in_specs=[
            pl.BlockSpec(memory_space=pltpu.VMEM),
            pl.BlockSpec(memory_space=pl.ANY),
            pl.BlockSpec(memory_space=pl.ANY),
        ],
        out_specs=pl.BlockSpec(memory_space=pltpu.VMEM),
        scratch_shapes=[
            pltpu.VMEM((N_DEV, HL, SKV_LOC, DH), jnp.bfloat16),
            pltpu.VMEM((N_DEV, HL, SKV_LOC, DH), jnp.bfloat16),
            pltpu.VMEM((SQ // QB, SKV), jnp.float32),
            pltpu.SemaphoreType.DMA((2, 3, HL)),
            pltpu.SemaphoreType.DMA((2, 3, HL)),
            pltpu.SemaphoreType.DMA((2,)),
        ],
        compiler_params=pltpu.CompilerParams(
            collective_id=0, vmem_limit_bytes=60 * 1024 * 1024),
    )(q, kt, vt)

    out = pl.pallas_call(
        _out_body,
        out_shape=jax.ShapeDtypeStruct((SQ, DM), jnp.float32),
        in_specs=[
            pl.BlockSpec(memory_space=pltpu.VMEM),
            pl.BlockSpec(memory_space=pltpu.VMEM),
        ],
        out_specs=pl.BlockSpec(memory_space=pltpu.VMEM),
        scratch_shapes=[
            pltpu.VMEM((SQ, DM), jnp.float32),
            pltpu.VMEM((2, QR, DM), jnp.bfloat16),
            pltpu.SemaphoreType.DMA((2,)),
            pltpu.SemaphoreType.DMA((2,)),
        ],
        compiler_params=pltpu.CompilerParams(
            collective_id=1, vmem_limit_bytes=60 * 1024 * 1024),
    )(ctx, wob)

    return out[None]


# device time: 418875 ns/iter; 1.7574x vs baseline; 1.2314x over previous
import jax
import jax.numpy as jnp
from jax import lax
from jax.experimental import pallas as pl
from jax.experimental.pallas import tpu as pltpu

N_DEV = 4
SQ = 2048
SKV_LOC = 2048
SKV = 8192
HL = 8
DH = 128
DM = 1024
SCALE = 0.08838834764831843
QB = 64
NB = SKV_LOC // QB

_RES_CNT = {c: [sum(1 for b in range(NB) if (2 * c + b) % 3 == r)
                for r in range(3)] for c in (1, 2, 3)}
_RES_OFF = {c: [sum(_RES_CNT[c][:r]) * QB for r in range(3)]
            for c in (1, 2, 3)}
_QGROUPS = {m: [qb for qb in range(NB) if qb % 3 == m] for m in range(3)}


def _subsets(lst, k=4):
    return [lst[i:i + k] for i in range(0, len(lst), k)]


def _attn_body(q_ref, kt_ref, vt_ref, ctx_ref,
               k_all, v_all, bias_ref, send_sems, recv_sems, copy_sems):
    my = lax.axis_index("i")

    bsem = pltpu.get_barrier_semaphore()
    for o in (1, 2, 3):
        pl.semaphore_signal(
            bsem, inc=1,
            device_id=((my + o) % N_DEV,),
            device_id_type=pl.DeviceIdType.MESH,
        )
    pl.semaphore_wait(bsem, 3)

    kc = pltpu.make_async_copy(kt_ref.at[my], k_all.at[my], copy_sems.at[0])
    vc = pltpu.make_async_copy(vt_ref.at[my], v_all.at[my], copy_sems.at[1])
    kc.start()
    vc.start()

    per_head = [[] for _ in range(HL)]
    for h in range(HL):
        for o in (1, 2, 3):
            peer = (my + o) % N_DEV
            for t, (src, dst) in enumerate(((kt_ref, k_all), (vt_ref, v_all))):
                r = pltpu.make_async_remote_copy(
                    src_ref=src.at[peer, h],
                    dst_ref=dst.at[my, h],
                    send_sem=send_sems.at[t, o - 1, h],
                    recv_sem=recv_sems.at[t, o - 1, h],
                    device_id=(peer,),
                    device_id_type=pl.DeviceIdType.MESH,
                )
                r.start()
                per_head[h].append(r)

    qb_ = lax.broadcasted_iota(jnp.int32, (NB, SKV_LOC), 0)
    kb_ = lax.broadcasted_iota(jnp.int32, (NB, SKV_LOC), 1) // QB
    keep = (qb_ == kb_) | (kb_ == 0) | (((qb_ + kb_) % 3) == 0)
    bias_ref[...] = jnp.where(keep, 0.0, -1e9).astype(jnp.float32)

    kc.wait()
    vc.wait()

    def h_step(h, _):
        for o in (1, 2, 3):
            peer = (my + o) % N_DEV
            for t, (src, dst) in enumerate(((kt_ref, k_all), (vt_ref, v_all))):
                rec = pltpu.make_async_remote_copy(
                    src_ref=src.at[peer, h],
                    dst_ref=dst.at[my, h],
                    send_sem=send_sems.at[t, o - 1, h],
                    recv_sem=recv_sems.at[t, o - 1, h],
                    device_id=(peer,),
                    device_id_type=pl.DeviceIdType.MESH,
                )
                rec.wait_recv()

        k0 = k_all[0, h]
        v0 = v_all[0, h]
        for m in range(3):
            r = (3 - m) % 3
            kr = jnp.concatenate(
                [k_all[c, h, _RES_OFF[c][r]:
                       _RES_OFF[c][r] + _RES_CNT[c][r] * QB, :]
                 for c in (1, 2, 3)], axis=0)
            vr = jnp.concatenate(
                [v_all[c, h, _RES_OFF[c][r]:
                       _RES_OFF[c][r] + _RES_CNT[c][r] * QB, :]
                 for c in (1, 2, 3)], axis=0)
            kcat = jnp.concatenate([k0, kr], axis=0)
            vcat = jnp.concatenate([v0, vr], axis=0)
            for qbs in _subsets(_QGROUPS[m]):
                rows = len(qbs) * QB
                qm = jnp.concatenate(
                    [q_ref[pl.ds(qb * QB, QB), pl.ds(h * DH, DH)]
                     for qb in qbs], axis=0)
                s = lax.dot_general(
                    qm, kcat, (((1,), (1,)), ((), ())),
                    preferred_element_type=jnp.float32)
                bias = jnp.concatenate(
                    [jnp.broadcast_to(bias_ref[qb, :], (QB, SKV_LOC))
                     for qb in qbs], axis=0)
                s = jnp.concatenate(
                    [s[:, :SKV_LOC] + bias, s[:, SKV_LOC:]], axis=1)
                mx = jnp.max(s, axis=1, keepdims=True)
                w = jnp.exp(s - mx)
                l = jnp.sum(w, axis=1, keepdims=True)
                p = (w / l).astype(jnp.bfloat16)
                cc = jnp.dot(p, vcat,
                             preferred_element_type=jnp.float32
                             ).astype(jnp.bfloat16)
                for i, qb in enumerate(qbs):
                    ctx_ref[pl.ds(qb * QB, QB), pl.ds(h * DH, DH)] = (
                        cc[i * QB:(i + 1) * QB, :])
        return 0

    lax.fori_loop(0, HL, h_step, 0)

    for hh in range(HL):
        for r in per_head[hh]:
            r.wait_send()


QR = SQ // N_DEV


def _out_body(ctx_ref, wo_ref, out_ref, par_ref, comm, send_sems, recv_sems):
    my = lax.axis_index("i")
    left = (my - 1) % N_DEV
    right = (my + 1) % N_DEV

    par_ref[...] = jnp.dot(ctx_ref[...], wo_ref[...],
                           preferred_element_type=jnp.float32)

    bsem = pltpu.get_barrier_semaphore()
    for nbr in (left, right):
        pl.semaphore_signal(
            bsem, inc=1,
            device_id=(nbr,), device_id_type=pl.DeviceIdType.MESH,
        )
    pl.semaphore_wait(bsem, 2)

    comm[0] = par_ref[pl.ds(my * QR, QR), :].astype(jnp.bfloat16)

    def hop(n):
        s_slot = n % 2
        r_slot = (n + 1) % 2
        r = pltpu.make_async_remote_copy(
            src_ref=comm.at[s_slot],
            dst_ref=comm.at[r_slot],
            send_sem=send_sems.at[s_slot],
            recv_sem=recv_sems.at[r_slot],
            device_id=(right,),
            device_id_type=pl.DeviceIdType.MESH,
        )
        r.start()
        r.wait()
        return r_slot

    for t in range(N_DEV - 1):
        r_slot = hop(t)
        rq = (my - 1 - t) % N_DEV
        val = comm[r_slot].astype(jnp.float32) + par_ref[pl.ds(rq * QR, QR), :]
        comm[r_slot] = val.astype(jnp.bfloat16)
        if t == N_DEV - 2:
            out_ref[pl.ds(((my + 1) % N_DEV) * QR, QR), :] = val

    for u in range(N_DEV - 1):
        r_slot = hop(N_DEV - 1 + u)
        q_idx = (my - u) % N_DEV
        out_ref[pl.ds(q_idx * QR, QR), :] = comm[r_slot].astype(jnp.float32)


def kernel(x, Wq, K_ext, V_ext, Wo):
    xb = x[0].astype(jnp.bfloat16)
    wqb = Wq.astype(jnp.bfloat16)
    q = (jnp.dot(xb, wqb) * SCALE).astype(jnp.bfloat16)

    my = lax.axis_index("i")
    b = jnp.arange(NB)
    perm = jnp.argsort((2 * my + b) % 3, stable=True)
    rows = (perm[:, None] * QB + jnp.arange(QB)[None, :]).reshape(-1)
    rows = jnp.where(my == 0, jnp.arange(SKV_LOC), rows)

    kt = (K_ext[0].astype(jnp.bfloat16)[rows]
          .reshape(SKV_LOC, N_DEV, HL, DH).transpose(1, 2, 0, 3))
    vt = (V_ext[0].astype(jnp.bfloat16)[rows]
          .reshape(SKV_LOC, N_DEV, HL, DH).transpose(1, 2, 0, 3))
    wob = Wo.astype(jnp.bfloat16)

    ctx = pl.pallas_call(
        _attn_body,
        out_shape=jax.ShapeDtypeStruct((SQ, HL * DH), jnp.bfloat16),
        in_specs=[
            pl.BlockSpec(memory_space=pltpu.VMEM),
            pl.BlockSpec(memory_space=pl.ANY),
            pl.BlockSpec(memory_space=pl.ANY),
        ],
        out_specs=pl.BlockSpec(memory_space=pltpu.VMEM),
        scratch_shapes=[
            pltpu.VMEM((N_DEV, HL, SKV_LOC, DH), jnp.bfloat16),
            pltpu.VMEM((N_DEV, HL, SKV_LOC, DH), jnp.bfloat16),
            pltpu.VMEM((NB, SKV_LOC), jnp.float32),
            pltpu.SemaphoreType.DMA((2, 3, HL)),
            pltpu.SemaphoreType.DMA((2, 3, HL)),
            pltpu.SemaphoreType.DMA((2,)),
        ],
        compiler_params=pltpu.CompilerParams(
            collective_id=0, vmem_limit_bytes=60 * 1024 * 1024),
    )(q, kt, vt)

    out = pl.pallas_call(
        _out_body,
        out_shape=jax.ShapeDtypeStruct((SQ, DM), jnp.float32),
        in_specs=[
            pl.BlockSpec(memory_space=pltpu.VMEM),
            pl.BlockSpec(memory_space=pltpu.VMEM),
        ],
        out_specs=pl.BlockSpec(memory_space=pltpu.VMEM),
        scratch_shapes=[
            pltpu.VMEM((SQ, DM), jnp.float32),
            pltpu.VMEM((2, QR, DM), jnp.bfloat16),
            pltpu.SemaphoreType.DMA((2,)),
            pltpu.SemaphoreType.DMA((2,)),
        ],
        compiler_params=pltpu.CompilerParams(
            collective_id=1, vmem_limit_bytes=60 * 1024 * 1024),
    )(ctx, wob)

    return out[None]


# device time: 377818 ns/iter; 1.9484x vs baseline; 1.1087x over previous
import jax
import jax.numpy as jnp
from jax import lax
from jax.experimental import pallas as pl
from jax.experimental.pallas import tpu as pltpu

N_DEV = 4
SQ = 2048
SKV_LOC = 2048
SKV = 8192
HL = 8
DH = 128
DM = 1024
SCALE = 0.08838834764831843
QB = 64
NB = SKV_LOC // QB

K_DTYPE = jnp.bfloat16
V_DTYPE = jnp.bfloat16

_RES_BLOCKS = {c: {r: list(range((r - 2 * c) % 3, NB, 3)) for r in range(3)}
               for c in (1, 2, 3)}
_QGROUPS = {m: [qb for qb in range(NB) if qb % 3 == m] for m in range(3)}


def _subsets(lst, k=4):
    return [lst[i:i + k] for i in range(0, len(lst), k)]


def _attn_body(x_ref, wq_ref, kt_ref, vt_ref, ctx_ref,
               k_all, v_all, q_scr, bias_ref,
               send_sems, recv_sems, copy_sems):
    my = lax.axis_index("i")

    bsem = pltpu.get_barrier_semaphore()
    for o in (1, 2, 3):
        pl.semaphore_signal(
            bsem, inc=1,
            device_id=((my + o) % N_DEV,),
            device_id_type=pl.DeviceIdType.MESH,
        )
    pl.semaphore_wait(bsem, 3)

    kc = pltpu.make_async_copy(kt_ref.at[my], k_all.at[my], copy_sems.at[0])
    vc = pltpu.make_async_copy(vt_ref.at[my], v_all.at[my], copy_sems.at[1])
    kc.start()
    vc.start()

    per_head = [[] for _ in range(HL)]
    for h in range(HL):
        for o in (1, 2, 3):
            peer = (my + o) % N_DEV
            for t, (src, dst) in enumerate(((kt_ref, k_all), (vt_ref, v_all))):
                r = pltpu.make_async_remote_copy(
                    src_ref=src.at[peer, h],
                    dst_ref=dst.at[my, h],
                    send_sem=send_sems.at[t, o - 1, h],
                    recv_sem=recv_sems.at[t, o - 1, h],
                    device_id=(peer,),
                    device_id_type=pl.DeviceIdType.MESH,
                )
                r.start()
                per_head[h].append(r)

    qb_ = lax.broadcasted_iota(jnp.int32, (NB, SKV_LOC), 0)
    kb_ = lax.broadcasted_iota(jnp.int32, (NB, SKV_LOC), 1) // QB
    keep = (qb_ == kb_) | (kb_ == 0) | (((qb_ + kb_) % 3) == 0)
    bias_ref[...] = jnp.where(keep, 0.0, -1e9).astype(jnp.float32)

    q_scr[...] = (lax.dot_general(
        x_ref[...], wq_ref[...], (((1,), (0,)), ((), ())),
        preferred_element_type=jnp.float32) * SCALE).astype(jnp.bfloat16)

    kc.wait()
    vc.wait()

    def h_step(h, _):
        for o in (1, 2, 3):
            peer = (my + o) % N_DEV
            for t, (src, dst) in enumerate(((kt_ref, k_all), (vt_ref, v_all))):
                rec = pltpu.make_async_remote_copy(
                    src_ref=src.at[peer, h],
                    dst_ref=dst.at[my, h],
                    send_sem=send_sems.at[t, o - 1, h],
                    recv_sem=recv_sems.at[t, o - 1, h],
                    device_id=(peer,),
                    device_id_type=pl.DeviceIdType.MESH,
                )
                rec.wait_recv()

        kbf = [k_all[c, h].astype(jnp.bfloat16) for c in range(N_DEV)]
        vbf = [v_all[c, h].astype(jnp.bfloat16) for c in range(N_DEV)]
        for m in range(3):
            r = (3 - m) % 3
            kcat = jnp.concatenate(
                [kbf[0]] + [kbf[c][b * QB:(b + 1) * QB, :]
                            for c in (1, 2, 3) for b in _RES_BLOCKS[c][r]],
                axis=0)
            vcat = jnp.concatenate(
                [vbf[0]] + [vbf[c][b * QB:(b + 1) * QB, :]
                            for c in (1, 2, 3) for b in _RES_BLOCKS[c][r]],
                axis=0)
            for qbs in _subsets(_QGROUPS[m]):
                rows = len(qbs) * QB
                qm = jnp.concatenate(
                    [q_scr[pl.ds(qb * QB, QB), pl.ds(h * DH, DH)]
                     for qb in qbs], axis=0)
                s = lax.dot_general(
                    qm, kcat, (((1,), (1,)), ((), ())),
                    preferred_element_type=jnp.float32)
                bias = jnp.concatenate(
                    [jnp.broadcast_to(bias_ref[qb, :], (QB, SKV_LOC))
                     for qb in qbs], axis=0)
                s = jnp.concatenate(
                    [s[:, :SKV_LOC] + bias, s[:, SKV_LOC:]], axis=1)
                mx = jnp.max(s, axis=1, keepdims=True)
                w = jnp.exp(s - mx)
                l = jnp.sum(w, axis=1, keepdims=True)
                p = (w / l).astype(jnp.bfloat16)
                cc = jnp.dot(p, vcat,
                             preferred_element_type=jnp.float32
                             ).astype(jnp.bfloat16)
                for i, qb in enumerate(qbs):
                    ctx_ref[pl.ds(qb * QB, QB), pl.ds(h * DH, DH)] = (
                        cc[i * QB:(i + 1) * QB, :])
        return 0

    lax.fori_loop(0, HL, h_step, 0)

    for hh in range(HL):
        for r in per_head[hh]:
            r.wait_send()


QR = SQ // N_DEV


def _out_body(ctx_ref, wo_ref, out_ref, par_ref, comm, send_sems, recv_sems):
    my = lax.axis_index("i")
    left = (my - 1) % N_DEV
    right = (my + 1) % N_DEV

    par_ref[...] = jnp.dot(ctx_ref[...], wo_ref[...],
                           preferred_element_type=jnp.float32)

    bsem = pltpu.get_barrier_semaphore()
    for nbr in (left, right):
        pl.semaphore_signal(
            bsem, inc=1,
            device_id=(nbr,), device_id_type=pl.DeviceIdType.MESH,
        )
    pl.semaphore_wait(bsem, 2)

    comm[0] = par_ref[pl.ds(my * QR, QR), :].astype(jnp.bfloat16)

    def hop(n):
        s_slot = n % 2
        r_slot = (n + 1) % 2
        r = pltpu.make_async_remote_copy(
            src_ref=comm.at[s_slot],
            dst_ref=comm.at[r_slot],
            send_sem=send_sems.at[s_slot],
            recv_sem=recv_sems.at[r_slot],
            device_id=(right,),
            device_id_type=pl.DeviceIdType.MESH,
        )
        r.start()
        r.wait()
        return r_slot

    for t in range(N_DEV - 1):
        r_slot = hop(t)
        rq = (my - 1 - t) % N_DEV
        val = comm[r_slot].astype(jnp.float32) + par_ref[pl.ds(rq * QR, QR), :]
        comm[r_slot] = val.astype(jnp.bfloat16)
        if t == N_DEV - 2:
            out_ref[pl.ds(((my + 1) % N_DEV) * QR, QR), :] = val

    for u in range(N_DEV - 1):
        r_slot = hop(N_DEV - 1 + u)
        q_idx = (my - u) % N_DEV
        out_ref[pl.ds(q_idx * QR, QR), :] = comm[r_slot].astype(jnp.float32)


def kernel(x, Wq, K_ext, V_ext, Wo):
    xb = x[0].astype(jnp.bfloat16)
    wqb = Wq.astype(jnp.bfloat16)

    kt = (K_ext[0].astype(K_DTYPE)
          .reshape(SKV_LOC, N_DEV, HL, DH).transpose(1, 2, 0, 3))
    vt = (V_ext[0].astype(V_DTYPE)
          .reshape(SKV_LOC, N_DEV, HL, DH).transpose(1, 2, 0, 3))
    wob = Wo.astype(jnp.bfloat16)

    ctx = pl.pallas_call(
        _attn_body,
        out_shape=jax.ShapeDtypeStruct((SQ, HL * DH), jnp.bfloat16),
        in_specs=[
            pl.BlockSpec(memory_space=pltpu.VMEM),
            pl.BlockSpec(memory_space=pltpu.VMEM),
            pl.BlockSpec(memory_space=pl.ANY),
            pl.BlockSpec(memory_space=pl.ANY),
        ],
        out_specs=pl.BlockSpec(memory_space=pltpu.VMEM),
        scratch_shapes=[
            pltpu.VMEM((N_DEV, HL, SKV_LOC, DH), K_DTYPE),
            pltpu.VMEM((N_DEV, HL, SKV_LOC, DH), V_DTYPE),
            pltpu.VMEM((SQ, HL * DH), jnp.bfloat16),
            pltpu.VMEM((NB, SKV_LOC), jnp.float32),
            pltpu.SemaphoreType.DMA((2, 3, HL)),
            pltpu.SemaphoreType.DMA((2, 3, HL)),
            pltpu.SemaphoreType.DMA((2,)),
        ],
        compiler_params=pltpu.CompilerParams(
            collective_id=0, vmem_limit_bytes=60 * 1024 * 1024),
    )(xb, wqb, kt, vt)

    out = pl.pallas_call(
        _out_body,
        out_shape=jax.ShapeDtypeStruct((SQ, DM), jnp.float32),
        in_specs=[
            pl.BlockSpec(memory_space=pltpu.VMEM),
            pl.BlockSpec(memory_space=pltpu.VMEM),
        ],
        out_specs=pl.BlockSpec(memory_space=pltpu.VMEM),
        scratch_shapes=[
            pltpu.VMEM((SQ, DM), jnp.float32),
            pltpu.VMEM((2, QR, DM), jnp.bfloat16),
            pltpu.SemaphoreType.DMA((2,)),
            pltpu.SemaphoreType.DMA((2,)),
        ],
        compiler_params=pltpu.CompilerParams(
            collective_id=1, vmem_limit_bytes=60 * 1024 * 1024),
    )(ctx, wob)

    return out[None]


# device time: 365421 ns/iter; 2.0145x vs baseline; 1.0339x over previous
import jax
import jax.numpy as jnp
from jax import lax
from jax.experimental import pallas as pl
from jax.experimental.pallas import tpu as pltpu

N_DEV = 4
SQ = 2048
SKV_LOC = 2048
SKV = 8192
HL = 8
DH = 128
DM = 1024
SCALE = 0.08838834764831843
QB = 64
NB = SKV_LOC // QB

K_DTYPE = jnp.bfloat16
V_DTYPE = jnp.bfloat16

_RES_BLOCKS = {c: {r: list(range((r - 2 * c) % 3, NB, 3)) for r in range(3)}
               for c in (1, 2, 3)}
_QGROUPS = {m: [qb for qb in range(NB) if qb % 3 == m] for m in range(3)}


def _subsets(lst, k=4):
    return [lst[i:i + k] for i in range(0, len(lst), k)]


def _attn_body(x_ref, wq_ref, kt_ref, vt_ref, ctx_ref,
               k_all, v_all, q_scr, bias_ref,
               send_sems, recv_sems, copy_sems):
    my = lax.axis_index("i")

    bsem = pltpu.get_barrier_semaphore()
    for o in (1, 2, 3):
        pl.semaphore_signal(
            bsem, inc=1,
            device_id=((my + o) % N_DEV,),
            device_id_type=pl.DeviceIdType.MESH,
        )
    pl.semaphore_wait(bsem, 3)

    kc = pltpu.make_async_copy(kt_ref.at[my], k_all.at[my], copy_sems.at[0])
    vc = pltpu.make_async_copy(vt_ref.at[my], v_all.at[my], copy_sems.at[1])
    kc.start()
    vc.start()

    per_head = [[] for _ in range(HL)]
    for h in range(HL):
        for o in (1, 2, 3):
            peer = (my + o) % N_DEV
            for t, (src, dst) in enumerate(((kt_ref, k_all), (vt_ref, v_all))):
                r = pltpu.make_async_remote_copy(
                    src_ref=src.at[peer, h],
                    dst_ref=dst.at[my, h],
                    send_sem=send_sems.at[t, o - 1, h],
                    recv_sem=recv_sems.at[t, o - 1, h],
                    device_id=(peer,),
                    device_id_type=pl.DeviceIdType.MESH,
                )
                r.start()
                per_head[h].append(r)

    qb_ = lax.broadcasted_iota(jnp.int32, (NB, SKV_LOC), 0)
    kb_ = lax.broadcasted_iota(jnp.int32, (NB, SKV_LOC), 1) // QB
    keep = (qb_ == kb_) | (kb_ == 0) | (((qb_ + kb_) % 3) == 0)
    bias_ref[...] = jnp.where(keep, 0.0, -1e9).astype(jnp.float32)

    q_scr[...] = (lax.dot_general(
        x_ref[...], wq_ref[...], (((1,), (0,)), ((), ())),
        preferred_element_type=jnp.float32) * SCALE).astype(jnp.bfloat16)

    kc.wait()
    vc.wait()

    def h_step(h, _):
        for o in (1, 2, 3):
            peer = (my + o) % N_DEV
            for t, (src, dst) in enumerate(((kt_ref, k_all), (vt_ref, v_all))):
                rec = pltpu.make_async_remote_copy(
                    src_ref=src.at[peer, h],
                    dst_ref=dst.at[my, h],
                    send_sem=send_sems.at[t, o - 1, h],
                    recv_sem=recv_sems.at[t, o - 1, h],
                    device_id=(peer,),
                    device_id_type=pl.DeviceIdType.MESH,
                )
                rec.wait_recv()

        kbf = [k_all[c, h].astype(jnp.bfloat16) for c in range(N_DEV)]
        vbf = [v_all[c, h].astype(jnp.bfloat16) for c in range(N_DEV)]
        for m in range(3):
            r = (3 - m) % 3
            kcat = jnp.concatenate(
                [kbf[0]] + [kbf[c][b * QB:(b + 1) * QB, :]
                            for c in (1, 2, 3) for b in _RES_BLOCKS[c][r]],
                axis=0)
            vcat = jnp.concatenate(
                [vbf[0]] + [vbf[c][b * QB:(b + 1) * QB, :]
                            for c in (1, 2, 3) for b in _RES_BLOCKS[c][r]],
                axis=0)
            for qbs in _subsets(_QGROUPS[m]):
                rows = len(qbs) * QB
                qm = jnp.concatenate(
                    [q_scr[pl.ds(qb * QB, QB), pl.ds(h * DH, DH)]
                     for qb in qbs], axis=0)
                s = lax.dot_general(
                    qm, kcat, (((1,), (1,)), ((), ())),
                    preferred_element_type=jnp.float32)
                bias = jnp.concatenate(
                    [jnp.broadcast_to(bias_ref[qb, :], (QB, SKV_LOC))
                     for qb in qbs], axis=0)
                s = jnp.concatenate(
                    [s[:, :SKV_LOC] + bias, s[:, SKV_LOC:]], axis=1)
                mx = jnp.max(s, axis=1, keepdims=True)
                w = jnp.exp((s - mx).astype(jnp.bfloat16))
                l = jnp.sum(w, axis=1, keepdims=True,
                            dtype=jnp.float32)
                p = w * (1.0 / l).astype(jnp.bfloat16)
                cc = jnp.dot(p, vcat,
                             preferred_element_type=jnp.float32
                             ).astype(jnp.bfloat16)
                for i, qb in enumerate(qbs):
                    ctx_ref[pl.ds(qb * QB, QB), pl.ds(h * DH, DH)] = (
                        cc[i * QB:(i + 1) * QB, :])
        return 0

    lax.fori_loop(0, HL, h_step, 0)

    for hh in range(HL):
        for r in per_head[hh]:
            r.wait_send()


QR = SQ // N_DEV


def _out_body(ctx_ref, wo_ref, out_ref, par_ref, comm, ag_buf,
              send_sems, recv_sems, ag_send_sems, ag_recv_sems):
    my = lax.axis_index("i")
    right = (my + 1) % N_DEV

    par_ref[...] = jnp.dot(ctx_ref[...], wo_ref[...],
                           preferred_element_type=jnp.float32)

    bsem = pltpu.get_barrier_semaphore()
    for o in (1, 2, 3):
        pl.semaphore_signal(
            bsem, inc=1,
            device_id=((my + o) % N_DEV,),
            device_id_type=pl.DeviceIdType.MESH,
        )
    pl.semaphore_wait(bsem, 3)

    comm[0] = par_ref[pl.ds(my * QR, QR), :].astype(jnp.bfloat16)

    def hop(n):
        s_slot = n % 2
        r_slot = (n + 1) % 2
        r = pltpu.make_async_remote_copy(
            src_ref=comm.at[s_slot],
            dst_ref=comm.at[r_slot],
            send_sem=send_sems.at[s_slot],
            recv_sem=recv_sems.at[r_slot],
            device_id=(right,),
            device_id_type=pl.DeviceIdType.MESH,
        )
        r.start()
        r.wait()
        return r_slot

    last_slot = 0
    for t in range(N_DEV - 1):
        r_slot = hop(t)
        rq = (my - 1 - t) % N_DEV
        val = comm[r_slot].astype(jnp.float32) + par_ref[pl.ds(rq * QR, QR), :]
        comm[r_slot] = val.astype(jnp.bfloat16)
        last_slot = r_slot
        if t == N_DEV - 2:
            out_ref[pl.ds(((my + 1) % N_DEV) * QR, QR), :] = val

    ag = []
    for o in (1, 2, 3):
        r = pltpu.make_async_remote_copy(
            src_ref=comm.at[last_slot],
            dst_ref=ag_buf.at[o - 1],
            send_sem=ag_send_sems.at[o - 1],
            recv_sem=ag_recv_sems.at[o - 1],
            device_id=((my + o) % N_DEV,),
            device_id_type=pl.DeviceIdType.MESH,
        )
        r.start()
        ag.append(r)
    for o, r in zip((1, 2, 3), ag):
        r.wait()
        q_idx = (my - o + 1) % N_DEV
        out_ref[pl.ds(q_idx * QR, QR), :] = ag_buf[o - 1].astype(jnp.float32)


def kernel(x, Wq, K_ext, V_ext, Wo):
    xb = x[0].astype(jnp.bfloat16)
    wqb = Wq.astype(jnp.bfloat16)

    kt = (K_ext[0].astype(K_DTYPE)
          .reshape(SKV_LOC, N_DEV, HL, DH).transpose(1, 2, 0, 3))
    vt = (V_ext[0].astype(V_DTYPE)
          .reshape(SKV_LOC, N_DEV, HL, DH).transpose(1, 2, 0, 3))
    wob = Wo.astype(jnp.bfloat16)

    ctx = pl.pallas_call(
        _attn_body,
        out_shape=jax.ShapeDtypeStruct((SQ, HL * DH), jnp.bfloat16),
        in_specs=[
            pl.BlockSpec(memory_space=pltpu.VMEM),
            pl.BlockSpec(memory_space=pltpu.VMEM),
            pl.BlockSpec(memory_space=pl.ANY),
            pl.BlockSpec(memory_space=pl.ANY),
        ],
        out_specs=pl.BlockSpec(memory_space=pltpu.VMEM),
        scratch_shapes=[
            pltpu.VMEM((N_DEV, HL, SKV_LOC, DH), K_DTYPE),
            pltpu.VMEM((N_DEV, HL, SKV_LOC, DH), V_DTYPE),
            pltpu.VMEM((SQ, HL * DH), jnp.bfloat16),
            pltpu.VMEM((NB, SKV_LOC), jnp.float32),
            pltpu.SemaphoreType.DMA((2, 3, HL)),
            pltpu.SemaphoreType.DMA((2, 3, HL)),
            pltpu.SemaphoreType.DMA((2,)),
        ],
        compiler_params=pltpu.CompilerParams(
            collective_id=0, vmem_limit_bytes=60 * 1024 * 1024),
    )(xb, wqb, kt, vt)

    out = pl.pallas_call(
        _out_body,
        out_shape=jax.ShapeDtypeStruct((SQ, DM), jnp.float32),
        in_specs=[
            pl.BlockSpec(memory_space=pltpu.VMEM),
            pl.BlockSpec(memory_space=pltpu.VMEM),
        ],
        out_specs=pl.BlockSpec(memory_space=pltpu.VMEM),
        scratch_shapes=[
            pltpu.VMEM((SQ, DM), jnp.float32),
            pltpu.VMEM((2, QR, DM), jnp.bfloat16),
            pltpu.VMEM((3, QR, DM), jnp.bfloat16),
            pltpu.SemaphoreType.DMA((2,)),
            pltpu.SemaphoreType.DMA((2,)),
            pltpu.SemaphoreType.DMA((3,)),
            pltpu.SemaphoreType.DMA((3,)),
        ],
        compiler_params=pltpu.CompilerParams(
            collective_id=1, vmem_limit_bytes=60 * 1024 * 1024),
    )(ctx, wob)

    return out[None]
